# Initial kernel scaffold; baseline (speedup 1.0000x reference)
import jax
import jax.numpy as jnp
from jax.experimental import pallas as pl


def kernel(x, cond, edge_index, batch, global_features, W1a, b1a, W1b, b1b, W2a, b2a, W2b, b2b):
    raise NotImplementedError("write your pallas kernel here")


# speedup vs baseline: 13.8257x; 13.8257x over previous
import functools
import jax
import jax.numpy as jnp
from jax import lax
from jax.experimental import pallas as pl
from jax.experimental.pallas import tpu as pltpu
from jax.experimental.pallas import tpu_sc as plsc

N = 10000
E = 320000
D = 128
G = 64
CG = 32
NCORE = 2
NSUB = 16
NWORK = NCORE * NSUB
CH = 100
NCHUNK_TOT = E // CH
CPT = NCHUNK_TOT // NWORK
IB = 20
NBATCH = CPT // IB
ROWS_PT = N // NSUB
ZROWS = 25

RB = 1000
NBLK = N // RB


def _onehot_f32(b_idx):
    iota = lax.broadcasted_iota(jnp.int32, (RB, G), 1)
    return jnp.where(b_idx[:, None] == iota, 1.0, 0.0).astype(jnp.float32)


def _proj_body(x_ref, b_ref, cg_ref, wx_ref, wg_ref, o_ref):
    gp = jnp.dot(cg_ref[...], wg_ref[...], preferred_element_type=jnp.float32)
    oh = _onehot_f32(b_ref[0, 0, :])
    o_ref[...] = (
        jnp.dot(x_ref[...], wx_ref[...], preferred_element_type=jnp.float32)
        + jnp.dot(oh, gp, preferred_element_type=jnp.float32)
    )


def _mid_body(p_ref, a_ref, b_ref, cg_ref, ba_ref, wb_ref, bb_ref,
              wx2_ref, wg2_ref, o_ref):
    z = p_ref[...] + a_ref[0] + a_ref[1] + ba_ref[...]
    t = jnp.where(z >= 0, z, 0.01 * z)
    x1 = jnp.dot(t, wb_ref[...], preferred_element_type=jnp.float32) + bb_ref[...]
    gp2 = jnp.dot(cg_ref[...], wg2_ref[...], preferred_element_type=jnp.float32)
    oh = _onehot_f32(b_ref[0, 0, :])
    o_ref[...] = (
        jnp.dot(x1, wx2_ref[...], preferred_element_type=jnp.float32)
        + jnp.dot(oh, gp2, preferred_element_type=jnp.float32)
    )


def _final_body(p_ref, a_ref, ba_ref, wb_ref, bb_ref, o_ref):
    z = p_ref[...] + a_ref[0] + a_ref[1] + ba_ref[...]
    t = jnp.where(z >= 0, z, 0.01 * z)
    o_ref[...] = jnp.dot(t, wb_ref[...], preferred_element_type=jnp.float32) + bb_ref[...]


_row_spec = pl.BlockSpec((RB, D), lambda i: (i, 0))
_batch_spec = pl.BlockSpec((1, 1, RB), lambda i: (i, 0, 0))
_agg_spec = pl.BlockSpec((NCORE, RB, D), lambda i: (0, i, 0))


def _full_spec(r, c):
    return pl.BlockSpec((r, c), lambda i: (0, 0))


_proj_call = pl.pallas_call(
    _proj_body,
    grid=(NBLK,),
    in_specs=[_row_spec, _batch_spec, _full_spec(G, CG), _full_spec(D, D),
              _full_spec(CG, D)],
    out_specs=_row_spec,
    out_shape=jax.ShapeDtypeStruct((N, D), jnp.float32),
)

_mid_call = pl.pallas_call(
    _mid_body,
    grid=(NBLK,),
    in_specs=[_row_spec, _agg_spec, _batch_spec, _full_spec(G, CG),
              _full_spec(1, D), _full_spec(D, D), _full_spec(1, D),
              _full_spec(D, D), _full_spec(CG, D)],
    out_specs=_row_spec,
    out_shape=jax.ShapeDtypeStruct((N, D), jnp.float32),
)

_final_call = pl.pallas_call(
    _final_body,
    grid=(NBLK,),
    in_specs=[_row_spec, _agg_spec, _full_spec(1, D), _full_spec(D, D),
              _full_spec(1, D)],
    out_specs=_row_spec,
    out_shape=jax.ShapeDtypeStruct((N, D), jnp.float32),
)


@functools.cache
def _make_segsum_sc():
  mesh = plsc.VectorSubcoreMesh(core_axis_name="c", subcore_axis_name="s")

  @functools.partial(
      pl.kernel,
      out_type=jax.ShapeDtypeStruct((NCORE, N, D), jnp.float32),
      mesh=mesh,
      compiler_params=pltpu.CompilerParams(use_tc_tiling_on_sc=False),
      scratch_types=[
          pltpu.VMEM((IB, CH), jnp.int32),
          pltpu.VMEM((IB, CH), jnp.int32),
          pltpu.VMEM((CH, D), jnp.float32),
          pltpu.VMEM((CH, D), jnp.float32),
          pltpu.VMEM((ZROWS, D), jnp.float32),
          pltpu.VMEM_SHARED((N, D), jnp.float32),
          pltpu.SemaphoreType.DMA,
          pltpu.SemaphoreType.DMA,
      ],
  )
  def _segsum_sc(p_hbm, src_hbm, dst_hbm, out_hbm,
                 src_v, dst_v, rows0, rows1, zbuf, acc, sem0, sem1):
    c = lax.axis_index("c")
    s = lax.axis_index("s")
    tchunk0 = (c * NSUB + s) * CPT

    def _zrow(i, carry):
      for j in range(D // 16):
        zbuf[i, pl.ds(j * 16, 16)] = jnp.zeros((16,), jnp.float32)
      return carry

    lax.fori_loop(0, ZROWS, _zrow, 0)
    for t in range(ROWS_PT // ZROWS):
      pltpu.sync_copy(zbuf, acc.at[pl.ds(s * ROWS_PT + t * ZROWS, ZROWS)])
    plsc.subcore_barrier()

    rows = (rows0, rows1)
    sems = (sem0, sem1)

    def _batch(ib, carry):
      bchunk0 = tchunk0 + ib * IB
      pltpu.sync_copy(src_hbm.at[pl.ds(bchunk0, IB)], src_v)
      pltpu.sync_copy(dst_hbm.at[pl.ds(bchunk0, IB)], dst_v)

      pltpu.async_copy(p_hbm.at[src_v.at[0]], rows0, sem0)
      pltpu.async_copy(p_hbm.at[src_v.at[1]], rows1, sem1)

      def _pair(k2, carry2):
        for b in range(2):
          k = k2 * 2 + b
          pltpu.make_async_copy(p_hbm.at[src_v.at[k]], rows[b], sems[b]).wait()

          @pl.when(k + 2 < IB)
          def _start_next():
            pltpu.async_copy(p_hbm.at[src_v.at[k + 2]], rows[b], sems[b])

          pltpu.sync_copy(rows[b], acc.at[dst_v.at[k]], add=True)
        return carry2

      lax.fori_loop(0, IB // 2, _pair, 0)
      return carry

    lax.fori_loop(0, NBATCH, _batch, 0)
    plsc.subcore_barrier()

    pltpu.sync_copy(acc.at[pl.ds(s * ROWS_PT, ROWS_PT)],
                    out_hbm.at[c, pl.ds(s * ROWS_PT, ROWS_PT)])

  return _segsum_sc


def kernel(x, cond, edge_index, batch, global_features,
           W1a, b1a, W1b, b1b, W2a, b2a, W2b, b2b):
    src = edge_index[0].reshape(NCHUNK_TOT, CH)
    dst = edge_index[1].reshape(NCHUNK_TOT, CH)
    cg = jnp.concatenate([cond, global_features], axis=1)
    batch3 = batch.reshape(NBLK, 1, RB)
    b1a2 = b1a.reshape(1, D)
    b1b2 = b1b.reshape(1, D)
    b2a2 = b2a.reshape(1, D)
    b2b2 = b2b.reshape(1, D)

    segsum_sc = _make_segsum_sc()
    p1 = _proj_call(x, batch3, cg, W1a[:D], W1a[D:])
    agg1 = segsum_sc(p1, src, dst)
    p2 = _mid_call(p1, agg1, batch3, cg, b1a2, W1b, b1b2, W2a[:D], W2a[D:])
    agg2 = segsum_sc(p2, src, dst)
    return _final_call(p2, agg2, b2a2, W2b, b2b2)


# speedup vs baseline: 13.8463x; 1.0015x over previous
import functools
import jax
import jax.numpy as jnp
from jax import lax
from jax.experimental import pallas as pl
from jax.experimental.pallas import tpu as pltpu
from jax.experimental.pallas import tpu_sc as plsc

N = 10000
E = 320000
D = 128
G = 64
CG = 32
NCORE = 2
NSUB = 16
NWORK = NCORE * NSUB
CH = 100
NCHUNK_TOT = E // CH
CPT = NCHUNK_TOT // NWORK
IB = 20
NBATCH = CPT // IB
ROWS_PT = N // NSUB
ZROWS = 25

RB = 1000
NBLK = N // RB


def _onehot_f32(b_idx):
    iota = lax.broadcasted_iota(jnp.int32, (RB, G), 1)
    return jnp.where(b_idx[:, None] == iota, 1.0, 0.0).astype(jnp.float32)


def _proj_body(x_ref, b_ref, cond_ref, gf_ref, w1a_ref, w2a_ref,
               p1_ref, goh2_ref):
    cg = jnp.concatenate([cond_ref[...], gf_ref[...]], axis=1)
    gp1 = jnp.dot(cg, w1a_ref[D:], preferred_element_type=jnp.float32)
    gp2 = jnp.dot(cg, w2a_ref[D:], preferred_element_type=jnp.float32)
    oh = _onehot_f32(b_ref[0, 0, :])
    p1_ref[...] = (
        jnp.dot(x_ref[...], w1a_ref[:D], preferred_element_type=jnp.float32)
        + jnp.dot(oh, gp1, preferred_element_type=jnp.float32)
    )
    goh2_ref[...] = jnp.dot(oh, gp2, preferred_element_type=jnp.float32)


def _mid_body(p_ref, a_ref, goh2_ref, ba_ref, wb_ref, bb_ref, wx2_ref, o_ref):
    z = p_ref[...] + a_ref[0] + a_ref[1] + ba_ref[...][None, :]
    t = jnp.where(z >= 0, z, 0.01 * z)
    x1 = jnp.dot(t, wb_ref[...], preferred_element_type=jnp.float32) + bb_ref[...][None, :]
    o_ref[...] = (
        jnp.dot(x1, wx2_ref[:D], preferred_element_type=jnp.float32)
        + goh2_ref[...]
    )


def _final_body(p_ref, a_ref, ba_ref, wb_ref, bb_ref, o_ref):
    z = p_ref[...] + a_ref[0] + a_ref[1] + ba_ref[...][None, :]
    t = jnp.where(z >= 0, z, 0.01 * z)
    o_ref[...] = (jnp.dot(t, wb_ref[...], preferred_element_type=jnp.float32)
                  + bb_ref[...][None, :])


_row_spec = pl.BlockSpec((RB, D), lambda i: (i, 0))
_batch_spec = pl.BlockSpec((1, 1, RB), lambda i: (i, 0, 0))
_agg_spec = pl.BlockSpec((NCORE, RB, D), lambda i: (0, i, 0))


def _full_spec(r, c):
    return pl.BlockSpec((r, c), lambda i: (0, 0))


def _vec_spec():
    return pl.BlockSpec((D,), lambda i: (0,))


_nd_f32 = jax.ShapeDtypeStruct((N, D), jnp.float32)

_proj_call = pl.pallas_call(
    _proj_body,
    grid=(NBLK,),
    in_specs=[_row_spec, _batch_spec, _full_spec(G, 16), _full_spec(G, 16),
              _full_spec(D + CG, D), _full_spec(D + CG, D)],
    out_specs=[_row_spec, _row_spec],
    out_shape=[_nd_f32, _nd_f32],
)

_mid_call = pl.pallas_call(
    _mid_body,
    grid=(NBLK,),
    in_specs=[_row_spec, _agg_spec, _row_spec, _vec_spec(),
              _full_spec(D, D), _vec_spec(), _full_spec(D + CG, D)],
    out_specs=_row_spec,
    out_shape=_nd_f32,
)

_final_call = pl.pallas_call(
    _final_body,
    grid=(NBLK,),
    in_specs=[_row_spec, _agg_spec, _vec_spec(), _full_spec(D, D),
              _vec_spec()],
    out_specs=_row_spec,
    out_shape=_nd_f32,
)


@functools.cache
def _make_segsum_sc():
  mesh = plsc.VectorSubcoreMesh(core_axis_name="c", subcore_axis_name="s")

  @functools.partial(
      pl.kernel,
      out_type=jax.ShapeDtypeStruct((NCORE, N, D), jnp.float32),
      mesh=mesh,
      compiler_params=pltpu.CompilerParams(use_tc_tiling_on_sc=False),
      scratch_types=[
          pltpu.VMEM((IB, CH), jnp.int32),
          pltpu.VMEM((IB, CH), jnp.int32),
          pltpu.VMEM((CH, D), jnp.float32),
          pltpu.VMEM((CH, D), jnp.float32),
          pltpu.VMEM((ZROWS, D), jnp.float32),
          pltpu.VMEM_SHARED((N, D), jnp.float32),
          pltpu.SemaphoreType.DMA,
          pltpu.SemaphoreType.DMA,
      ],
  )
  def _segsum_sc(p_hbm, src_hbm, dst_hbm, out_hbm,
                 src_v, dst_v, rows0, rows1, zbuf, acc, sem0, sem1):
    c = lax.axis_index("c")
    s = lax.axis_index("s")
    tchunk0 = (c * NSUB + s) * CPT

    def _zrow(i, carry):
      for j in range(D // 16):
        zbuf[i, pl.ds(j * 16, 16)] = jnp.zeros((16,), jnp.float32)
      return carry

    lax.fori_loop(0, ZROWS, _zrow, 0)
    for t in range(ROWS_PT // ZROWS):
      pltpu.sync_copy(zbuf, acc.at[pl.ds(s * ROWS_PT + t * ZROWS, ZROWS)])
    plsc.subcore_barrier()

    rows = (rows0, rows1)
    sems = (sem0, sem1)

    def _batch(ib, carry):
      bchunk0 = tchunk0 + ib * IB
      pltpu.sync_copy(src_hbm.at[pl.ds(bchunk0, IB)], src_v)
      pltpu.sync_copy(dst_hbm.at[pl.ds(bchunk0, IB)], dst_v)

      pltpu.async_copy(p_hbm.at[src_v.at[0]], rows0, sem0)
      pltpu.async_copy(p_hbm.at[src_v.at[1]], rows1, sem1)

      def _pair(k2, carry2):
        for b in range(2):
          k = k2 * 2 + b
          pltpu.make_async_copy(p_hbm.at[src_v.at[k]], rows[b], sems[b]).wait()

          @pl.when(k + 2 < IB)
          def _start_next():
            pltpu.async_copy(p_hbm.at[src_v.at[k + 2]], rows[b], sems[b])

          pltpu.sync_copy(rows[b], acc.at[dst_v.at[k]], add=True)
        return carry2

      lax.fori_loop(0, IB // 2, _pair, 0)
      return carry

    lax.fori_loop(0, NBATCH, _batch, 0)
    plsc.subcore_barrier()

    pltpu.sync_copy(acc.at[pl.ds(s * ROWS_PT, ROWS_PT)],
                    out_hbm.at[c, pl.ds(s * ROWS_PT, ROWS_PT)])

  return _segsum_sc


def kernel(x, cond, edge_index, batch, global_features,
           W1a, b1a, W1b, b1b, W2a, b2a, W2b, b2b):
    src = edge_index[0].reshape(NCHUNK_TOT, CH)
    dst = edge_index[1].reshape(NCHUNK_TOT, CH)
    batch3 = batch.reshape(NBLK, 1, RB)

    segsum_sc = _make_segsum_sc()
    p1, goh2 = _proj_call(x, batch3, cond, global_features, W1a, W2a)
    agg1 = segsum_sc(p1, src, dst)
    p2 = _mid_call(p1, agg1, goh2, b1a, W1b, b1b, W2a)
    agg2 = segsum_sc(p2, src, dst)
    return _final_call(p2, agg2, b2a, W2b, b2b)


# speedup vs baseline: 14.2353x; 1.0281x over previous
import functools
import jax
import jax.numpy as jnp
from jax import lax
from jax.experimental import pallas as pl
from jax.experimental.pallas import tpu as pltpu
from jax.experimental.pallas import tpu_sc as plsc

N = 10000
E = 320000
D = 128
G = 64
CG = 32
NCORE = 2
NSUB = 16
NWORK = NCORE * NSUB
CH = 100
NCHUNK_TOT = E // CH
CPT = NCHUNK_TOT // NWORK
IB = 20
NBATCH = CPT // IB
ROWS_PT = N // NSUB
ZROWS = 25

RB = 1000
NBLK = N // RB


def _onehot_f32(b_idx):
    iota = lax.broadcasted_iota(jnp.int32, (RB, G), 1)
    return jnp.where(b_idx[:, None] == iota, 1.0, 0.0).astype(jnp.float32)


def _proj_body(x_ref, b_ref, cond_ref, gf_ref, w1a_ref, w2a_ref,
               p1_ref, goh2_ref):
    cg = jnp.concatenate([cond_ref[...], gf_ref[...]], axis=1)
    gp1 = jnp.dot(cg, w1a_ref[D:], preferred_element_type=jnp.float32)
    gp2 = jnp.dot(cg, w2a_ref[D:], preferred_element_type=jnp.float32)
    oh = _onehot_f32(b_ref[0, 0, :])
    p1_ref[...] = (
        jnp.dot(x_ref[...], w1a_ref[:D], preferred_element_type=jnp.float32)
        + jnp.dot(oh, gp1, preferred_element_type=jnp.float32)
    )
    goh2_ref[...] = jnp.dot(oh, gp2, preferred_element_type=jnp.float32)


def _mid_body(p_ref, a_ref, goh2_ref, ba_ref, wb_ref, bb_ref, wx2_ref, o_ref):
    z = p_ref[...] + a_ref[0] + a_ref[1] + ba_ref[...][None, :]
    t = jnp.where(z >= 0, z, 0.01 * z)
    x1 = jnp.dot(t, wb_ref[...], preferred_element_type=jnp.float32) + bb_ref[...][None, :]
    o_ref[...] = (
        jnp.dot(x1, wx2_ref[:D], preferred_element_type=jnp.float32)
        + goh2_ref[...]
    )


def _final_body(p_ref, a_ref, ba_ref, wb_ref, bb_ref, o_ref):
    z = p_ref[...] + a_ref[0] + a_ref[1] + ba_ref[...][None, :]
    t = jnp.where(z >= 0, z, 0.01 * z)
    o_ref[...] = (jnp.dot(t, wb_ref[...], preferred_element_type=jnp.float32)
                  + bb_ref[...][None, :])


_row_spec = pl.BlockSpec((RB, D), lambda i: (i, 0))
_batch_spec = pl.BlockSpec((1, 1, RB), lambda i: (i, 0, 0))
_agg_spec = pl.BlockSpec((NCORE, RB, D), lambda i: (0, i, 0))


def _full_spec(r, c):
    return pl.BlockSpec((r, c), lambda i: (0, 0))


def _vec_spec():
    return pl.BlockSpec((D,), lambda i: (0,))


_nd_f32 = jax.ShapeDtypeStruct((N, D), jnp.float32)

_proj_call = pl.pallas_call(
    _proj_body,
    grid=(NBLK,),
    in_specs=[_row_spec, _batch_spec, _full_spec(G, 16), _full_spec(G, 16),
              _full_spec(D + CG, D), _full_spec(D + CG, D)],
    out_specs=[_row_spec, _row_spec],
    out_shape=[_nd_f32, _nd_f32],
)

_mid_call = pl.pallas_call(
    _mid_body,
    grid=(NBLK,),
    in_specs=[_row_spec, _agg_spec, _row_spec, _vec_spec(),
              _full_spec(D, D), _vec_spec(), _full_spec(D + CG, D)],
    out_specs=_row_spec,
    out_shape=_nd_f32,
)

_final_call = pl.pallas_call(
    _final_body,
    grid=(NBLK,),
    in_specs=[_row_spec, _agg_spec, _vec_spec(), _full_spec(D, D),
              _vec_spec()],
    out_specs=_row_spec,
    out_shape=_nd_f32,
)


@functools.cache
def _make_segsum_sc():
  mesh = plsc.VectorSubcoreMesh(core_axis_name="c", subcore_axis_name="s")

  @functools.partial(
      pl.kernel,
      out_type=jax.ShapeDtypeStruct((NCORE, N, D), jnp.float32),
      mesh=mesh,
      compiler_params=pltpu.CompilerParams(use_tc_tiling_on_sc=False,
                                           disable_bounds_checks=True),
      scratch_types=[
          pltpu.VMEM((IB, CH), jnp.int32),
          pltpu.VMEM((IB, CH), jnp.int32),
          pltpu.VMEM((CH, D), jnp.float32),
          pltpu.VMEM((CH, D), jnp.float32),
          pltpu.VMEM((ZROWS, D), jnp.float32),
          pltpu.VMEM_SHARED((N, D), jnp.float32),
          pltpu.SemaphoreType.DMA,
          pltpu.SemaphoreType.DMA,
      ],
  )
  def _segsum_sc(p_hbm, edges_hbm, out_hbm,
                 src_v, dst_v, rows0, rows1, zbuf, acc, sem0, sem1):
    c = lax.axis_index("c")
    s = lax.axis_index("s")
    tchunk0 = (c * NSUB + s) * CPT

    def _zrow(i, carry):
      for j in range(D // 16):
        zbuf[i, pl.ds(j * 16, 16)] = jnp.zeros((16,), jnp.float32)
      return carry

    lax.fori_loop(0, ZROWS, _zrow, 0)
    for t in range(ROWS_PT // ZROWS):
      pltpu.sync_copy(zbuf, acc.at[pl.ds(s * ROWS_PT + t * ZROWS, ZROWS)])
    plsc.subcore_barrier()

    rows = (rows0, rows1)
    sems = (sem0, sem1)

    def _batch(ib, carry):
      bchunk0 = tchunk0 + ib * IB
      pltpu.sync_copy(edges_hbm.at[0, pl.ds(bchunk0, IB)], src_v)
      pltpu.sync_copy(edges_hbm.at[1, pl.ds(bchunk0, IB)], dst_v)

      pltpu.async_copy(p_hbm.at[src_v.at[0]], rows0, sem0)
      pltpu.async_copy(p_hbm.at[src_v.at[1]], rows1, sem1)

      def _pair(k2, carry2):
        for b in range(2):
          k = k2 * 2 + b
          pltpu.make_async_copy(p_hbm.at[src_v.at[k]], rows[b], sems[b]).wait()

          @pl.when(k + 2 < IB)
          def _start_next():
            pltpu.async_copy(p_hbm.at[src_v.at[k + 2]], rows[b], sems[b])

          pltpu.sync_copy(rows[b], acc.at[dst_v.at[k]], add=True)
        return carry2

      lax.fori_loop(0, IB // 2, _pair, 0)
      return carry

    lax.fori_loop(0, NBATCH, _batch, 0)
    plsc.subcore_barrier()

    pltpu.sync_copy(acc.at[pl.ds(s * ROWS_PT, ROWS_PT)],
                    out_hbm.at[c, pl.ds(s * ROWS_PT, ROWS_PT)])

  return _segsum_sc


def kernel(x, cond, edge_index, batch, global_features,
           W1a, b1a, W1b, b1b, W2a, b2a, W2b, b2b):
    edges = edge_index.reshape(2, NCHUNK_TOT, CH)
    batch3 = batch.reshape(NBLK, 1, RB)

    segsum_sc = _make_segsum_sc()
    p1, goh2 = _proj_call(x, batch3, cond, global_features, W1a, W2a)
    agg1 = segsum_sc(p1, edges)
    p2 = _mid_call(p1, agg1, goh2, b1a, W1b, b1b, W2a)
    agg2 = segsum_sc(p2, edges)
    return _final_call(p2, agg2, b2a, W2b, b2b)


# speedup vs baseline: 15.5313x; 1.0910x over previous
import functools
import jax
import jax.numpy as jnp
from jax import lax
from jax.experimental import pallas as pl
from jax.experimental.pallas import tpu as pltpu
from jax.experimental.pallas import tpu_sc as plsc

N = 10000
E = 320000
D = 128
G = 64
CG = 32
NCORE = 2
NSUB = 16
NWORK = NCORE * NSUB
CH = 128
NCHUNK_TOT = E // CH
CPT = 78
NEXTRA = NCHUNK_TOT - CPT * NWORK
IB = 26
NBATCH = CPT // IB
ROWS_PT = N // NSUB
ZROWS = 125

RB = 1000
NBLK = N // RB


def _onehot_f32(b_idx):
    iota = lax.broadcasted_iota(jnp.int32, (RB, G), 1)
    return jnp.where(b_idx[:, None] == iota, 1.0, 0.0).astype(jnp.float32)


def _proj_body(x_ref, b_ref, cond_ref, gf_ref, w1a_ref, w2a_ref,
               p1_ref, goh2_ref):
    cg = jnp.concatenate([cond_ref[...], gf_ref[...]], axis=1)
    gp1 = jnp.dot(cg, w1a_ref[D:], preferred_element_type=jnp.float32)
    gp2 = jnp.dot(cg, w2a_ref[D:], preferred_element_type=jnp.float32)
    oh = _onehot_f32(b_ref[0, 0, :])
    p1_ref[...] = (
        jnp.dot(x_ref[...], w1a_ref[:D], preferred_element_type=jnp.float32)
        + jnp.dot(oh, gp1, preferred_element_type=jnp.float32)
    )
    goh2_ref[...] = jnp.dot(oh, gp2, preferred_element_type=jnp.float32)


def _mid_body(p_ref, a_ref, goh2_ref, ba_ref, wb_ref, bb_ref, wx2_ref, o_ref):
    z = p_ref[...] + a_ref[0] + a_ref[1] + ba_ref[...][None, :]
    t = jnp.where(z >= 0, z, 0.01 * z)
    x1 = jnp.dot(t, wb_ref[...], preferred_element_type=jnp.float32) + bb_ref[...][None, :]
    o_ref[...] = (
        jnp.dot(x1, wx2_ref[:D], preferred_element_type=jnp.float32)
        + goh2_ref[...]
    )


def _final_body(p_ref, a_ref, ba_ref, wb_ref, bb_ref, o_ref):
    z = p_ref[...] + a_ref[0] + a_ref[1] + ba_ref[...][None, :]
    t = jnp.where(z >= 0, z, 0.01 * z)
    o_ref[...] = (jnp.dot(t, wb_ref[...], preferred_element_type=jnp.float32)
                  + bb_ref[...][None, :])


_row_spec = pl.BlockSpec((RB, D), lambda i: (i, 0))
_batch_spec = pl.BlockSpec((1, 1, RB), lambda i: (i, 0, 0))
_agg_spec = pl.BlockSpec((NCORE, RB, D), lambda i: (0, i, 0))


def _full_spec(r, c):
    return pl.BlockSpec((r, c), lambda i: (0, 0))


def _vec_spec():
    return pl.BlockSpec((D,), lambda i: (0,))


_nd_f32 = jax.ShapeDtypeStruct((N, D), jnp.float32)

_proj_call = pl.pallas_call(
    _proj_body,
    grid=(NBLK,),
    in_specs=[_row_spec, _batch_spec, _full_spec(G, 16), _full_spec(G, 16),
              _full_spec(D + CG, D), _full_spec(D + CG, D)],
    out_specs=[_row_spec, _row_spec],
    out_shape=[_nd_f32, _nd_f32],
)

_mid_call = pl.pallas_call(
    _mid_body,
    grid=(NBLK,),
    in_specs=[_row_spec, _agg_spec, _row_spec, _vec_spec(),
              _full_spec(D, D), _vec_spec(), _full_spec(D + CG, D)],
    out_specs=_row_spec,
    out_shape=_nd_f32,
)

_final_call = pl.pallas_call(
    _final_body,
    grid=(NBLK,),
    in_specs=[_row_spec, _agg_spec, _vec_spec(), _full_spec(D, D),
              _vec_spec()],
    out_specs=_row_spec,
    out_shape=_nd_f32,
)


@functools.cache
def _make_segsum_sc():
  mesh = plsc.VectorSubcoreMesh(core_axis_name="c", subcore_axis_name="s")

  @functools.partial(
      pl.kernel,
      out_type=jax.ShapeDtypeStruct((NCORE, N, D), jnp.float32),
      mesh=mesh,
      compiler_params=pltpu.CompilerParams(use_tc_tiling_on_sc=False,
                                           disable_bounds_checks=True),
      scratch_types=[
          pltpu.VMEM((IB, CH), jnp.int32),
          pltpu.VMEM((IB, CH), jnp.int32),
          pltpu.VMEM((CH, D), jnp.float32),
          pltpu.VMEM((CH, D), jnp.float32),
          pltpu.VMEM_SHARED((N, D), jnp.float32),
          pltpu.SemaphoreType.DMA,
          pltpu.SemaphoreType.DMA,
      ],
  )
  def _segsum_sc(p_hbm, edges_hbm, out_hbm,
                 src_v, dst_v, rows0, rows1, acc, sem0, sem1):
    c = lax.axis_index("c")
    s = lax.axis_index("s")
    wid = c * NSUB + s
    tchunk0 = wid * CPT

    pltpu.async_copy(edges_hbm.at[0, pl.ds(tchunk0, IB)], src_v, sem0)
    pltpu.async_copy(edges_hbm.at[1, pl.ds(tchunk0, IB)], dst_v, sem1)

    def _zrow(i, carry):
      for j in range(D // 16):
        rows0[i, pl.ds(j * 16, 16)] = jnp.zeros((16,), jnp.float32)
      return carry

    lax.fori_loop(0, CH, _zrow, 0)
    for t in range(ROWS_PT // ZROWS):
      pltpu.sync_copy(rows0.at[pl.ds(0, ZROWS)],
                      acc.at[pl.ds(s * ROWS_PT + t * ZROWS, ZROWS)])
    pltpu.make_async_copy(edges_hbm.at[0, pl.ds(tchunk0, IB)], src_v, sem0).wait()
    pltpu.make_async_copy(edges_hbm.at[1, pl.ds(tchunk0, IB)], dst_v, sem1).wait()
    plsc.subcore_barrier()

    rows = (rows0, rows1)
    sems = (sem0, sem1)

    def _batch(ib, carry):
      bchunk0 = tchunk0 + ib * IB

      @pl.when(ib > 0)
      def _load_idx():
        pltpu.sync_copy(edges_hbm.at[0, pl.ds(bchunk0, IB)], src_v)
        pltpu.sync_copy(edges_hbm.at[1, pl.ds(bchunk0, IB)], dst_v)

      pltpu.async_copy(p_hbm.at[src_v.at[0]], rows0, sem0)
      pltpu.async_copy(p_hbm.at[src_v.at[1]], rows1, sem1)

      def _pair(k2, carry2):
        for b in range(2):
          k = k2 * 2 + b
          pltpu.make_async_copy(p_hbm.at[src_v.at[k]], rows[b], sems[b]).wait()

          @pl.when(k + 2 < IB)
          def _start_next():
            pltpu.async_copy(p_hbm.at[src_v.at[k + 2]], rows[b], sems[b])

          pltpu.sync_copy(rows[b], acc.at[dst_v.at[k]], add=True)
        return carry2

      lax.fori_loop(0, IB // 2, _pair, 0)
      return carry

    lax.fori_loop(0, NBATCH, _batch, 0)

    @pl.when(wid < NEXTRA)
    def _epilogue():
      kx = CPT * NWORK + wid
      pltpu.sync_copy(edges_hbm.at[0, pl.ds(kx, 1)], src_v.at[pl.ds(0, 1)])
      pltpu.sync_copy(edges_hbm.at[1, pl.ds(kx, 1)], dst_v.at[pl.ds(0, 1)])
      pltpu.async_copy(p_hbm.at[src_v.at[0]], rows0, sem0).wait()
      pltpu.sync_copy(rows0, acc.at[dst_v.at[0]], add=True)

    plsc.subcore_barrier()

    pltpu.sync_copy(acc.at[pl.ds(s * ROWS_PT, ROWS_PT)],
                    out_hbm.at[c, pl.ds(s * ROWS_PT, ROWS_PT)])

  return _segsum_sc


def kernel(x, cond, edge_index, batch, global_features,
           W1a, b1a, W1b, b1b, W2a, b2a, W2b, b2b):
    edges = edge_index.reshape(2, NCHUNK_TOT, CH)
    batch3 = batch.reshape(NBLK, 1, RB)

    segsum_sc = _make_segsum_sc()
    p1, goh2 = _proj_call(x, batch3, cond, global_features, W1a, W2a)
    agg1 = segsum_sc(p1, edges)
    p2 = _mid_call(p1, agg1, goh2, b1a, W1b, b1b, W2a)
    agg2 = segsum_sc(p2, edges)
    return _final_call(p2, agg2, b2a, W2b, b2b)


# speedup vs baseline: 15.6256x; 1.0061x over previous
import functools
import jax
import jax.numpy as jnp
from jax import lax
from jax.experimental import pallas as pl
from jax.experimental.pallas import tpu as pltpu
from jax.experimental.pallas import tpu_sc as plsc

N = 10000
E = 320000
D = 128
G = 64
CG = 32
NCORE = 2
NSUB = 16
NWORK = NCORE * NSUB
CH = 128
NCHUNK_TOT = E // CH
CPT = 78
NEXTRA = NCHUNK_TOT - CPT * NWORK
IB = 26
NBATCH = CPT // IB
ROWS_PT = N // NSUB
ZROWS = 125

RB = 1000
NBLK = N // RB


def _onehot_f32(b_idx):
    iota = lax.broadcasted_iota(jnp.int32, (RB, G), 1)
    return jnp.where(b_idx[:, None] == iota, 1.0, 0.0).astype(jnp.float32)


def _proj_body(x_ref, b_ref, cond_ref, gf_ref, w1a_ref, p1_ref):
    cg = jnp.concatenate([cond_ref[...], gf_ref[...]], axis=1)
    gp1 = jnp.dot(cg, w1a_ref[D:], preferred_element_type=jnp.float32)
    oh = _onehot_f32(b_ref[0, 0, :])
    p1_ref[...] = (
        jnp.dot(x_ref[...], w1a_ref[:D], preferred_element_type=jnp.float32)
        + jnp.dot(oh, gp1, preferred_element_type=jnp.float32)
    )


def _mid_body(p_ref, a_ref, b_ref, cond_ref, gf_ref, ba_ref, wb_ref, bb_ref,
              w2a_ref, o_ref):
    z = p_ref[...] + a_ref[0] + a_ref[1] + ba_ref[...][None, :]
    t = jnp.where(z >= 0, z, 0.01 * z)
    x1 = jnp.dot(t, wb_ref[...], preferred_element_type=jnp.float32) + bb_ref[...][None, :]
    cg = jnp.concatenate([cond_ref[...], gf_ref[...]], axis=1)
    gp2 = jnp.dot(cg, w2a_ref[D:], preferred_element_type=jnp.float32)
    oh = _onehot_f32(b_ref[0, 0, :])
    o_ref[...] = (
        jnp.dot(x1, w2a_ref[:D], preferred_element_type=jnp.float32)
        + jnp.dot(oh, gp2, preferred_element_type=jnp.float32)
    )


def _final_body(p_ref, a_ref, ba_ref, wb_ref, bb_ref, o_ref):
    z = p_ref[...] + a_ref[0] + a_ref[1] + ba_ref[...][None, :]
    t = jnp.where(z >= 0, z, 0.01 * z)
    o_ref[...] = (jnp.dot(t, wb_ref[...], preferred_element_type=jnp.float32)
                  + bb_ref[...][None, :])


_row_spec = pl.BlockSpec((RB, D), lambda i: (i, 0))
_batch_spec = pl.BlockSpec((1, 1, RB), lambda i: (i, 0, 0))
_agg_spec = pl.BlockSpec((NCORE, RB, D), lambda i: (0, i, 0))


def _full_spec(r, c):
    return pl.BlockSpec((r, c), lambda i: (0, 0))


def _vec_spec():
    return pl.BlockSpec((D,), lambda i: (0,))


_nd_f32 = jax.ShapeDtypeStruct((N, D), jnp.float32)

_proj_call = pl.pallas_call(
    _proj_body,
    grid=(NBLK,),
    in_specs=[_row_spec, _batch_spec, _full_spec(G, 16), _full_spec(G, 16),
              _full_spec(D + CG, D)],
    out_specs=_row_spec,
    out_shape=_nd_f32,
)

_mid_call = pl.pallas_call(
    _mid_body,
    grid=(NBLK,),
    in_specs=[_row_spec, _agg_spec, _batch_spec, _full_spec(G, 16),
              _full_spec(G, 16), _vec_spec(), _full_spec(D, D), _vec_spec(),
              _full_spec(D + CG, D)],
    out_specs=_row_spec,
    out_shape=_nd_f32,
)

_final_call = pl.pallas_call(
    _final_body,
    grid=(NBLK,),
    in_specs=[_row_spec, _agg_spec, _vec_spec(), _full_spec(D, D),
              _vec_spec()],
    out_specs=_row_spec,
    out_shape=_nd_f32,
)


@functools.cache
def _make_segsum_sc():
  mesh = plsc.VectorSubcoreMesh(core_axis_name="c", subcore_axis_name="s")

  @functools.partial(
      pl.kernel,
      out_type=jax.ShapeDtypeStruct((NCORE, N, D), jnp.float32),
      mesh=mesh,
      compiler_params=pltpu.CompilerParams(use_tc_tiling_on_sc=False,
                                           disable_bounds_checks=True),
      scratch_types=[
          pltpu.VMEM((IB, CH), jnp.int32),
          pltpu.VMEM((IB, CH), jnp.int32),
          pltpu.VMEM((CH, D), jnp.float32),
          pltpu.VMEM((CH, D), jnp.float32),
          pltpu.VMEM_SHARED((N, D), jnp.float32),
          pltpu.SemaphoreType.DMA,
          pltpu.SemaphoreType.DMA,
      ],
  )
  def _segsum_sc(p_hbm, edges_hbm, out_hbm,
                 src_v, dst_v, rows0, rows1, acc, sem0, sem1):
    c = lax.axis_index("c")
    s = lax.axis_index("s")
    wid = c * NSUB + s
    tchunk0 = wid * CPT

    pltpu.async_copy(edges_hbm.at[0, pl.ds(tchunk0, IB)], src_v, sem0)
    pltpu.async_copy(edges_hbm.at[1, pl.ds(tchunk0, IB)], dst_v, sem1)

    def _zrow(i, carry):
      for j in range(D // 16):
        rows0[i, pl.ds(j * 16, 16)] = jnp.zeros((16,), jnp.float32)
      return carry

    lax.fori_loop(0, CH, _zrow, 0)
    for t in range(ROWS_PT // ZROWS):
      pltpu.sync_copy(rows0.at[pl.ds(0, ZROWS)],
                      acc.at[pl.ds(s * ROWS_PT + t * ZROWS, ZROWS)])
    pltpu.make_async_copy(edges_hbm.at[0, pl.ds(tchunk0, IB)], src_v, sem0).wait()
    pltpu.make_async_copy(edges_hbm.at[1, pl.ds(tchunk0, IB)], dst_v, sem1).wait()
    plsc.subcore_barrier()

    rows = (rows0, rows1)
    sems = (sem0, sem1)

    def _batch(ib, carry):
      bchunk0 = tchunk0 + ib * IB

      @pl.when(ib > 0)
      def _load_idx():
        pltpu.sync_copy(edges_hbm.at[0, pl.ds(bchunk0, IB)], src_v)
        pltpu.sync_copy(edges_hbm.at[1, pl.ds(bchunk0, IB)], dst_v)

      pltpu.async_copy(p_hbm.at[src_v.at[0]], rows0, sem0)
      pltpu.async_copy(p_hbm.at[src_v.at[1]], rows1, sem1)

      def _pair(k2, carry2):
        for b in range(2):
          k = k2 * 2 + b
          pltpu.make_async_copy(p_hbm.at[src_v.at[k]], rows[b], sems[b]).wait()

          @pl.when(k + 2 < IB)
          def _start_next():
            pltpu.async_copy(p_hbm.at[src_v.at[k + 2]], rows[b], sems[b])

          pltpu.sync_copy(rows[b], acc.at[dst_v.at[k]], add=True)
        return carry2

      lax.fori_loop(0, IB // 2, _pair, 0)
      return carry

    lax.fori_loop(0, NBATCH, _batch, 0)

    @pl.when(wid < NEXTRA)
    def _epilogue():
      kx = CPT * NWORK + wid
      pltpu.sync_copy(edges_hbm.at[0, pl.ds(kx, 1)], src_v.at[pl.ds(0, 1)])
      pltpu.sync_copy(edges_hbm.at[1, pl.ds(kx, 1)], dst_v.at[pl.ds(0, 1)])
      pltpu.async_copy(p_hbm.at[src_v.at[0]], rows0, sem0).wait()
      pltpu.sync_copy(rows0, acc.at[dst_v.at[0]], add=True)

    plsc.subcore_barrier()

    pltpu.sync_copy(acc.at[pl.ds(s * ROWS_PT, ROWS_PT)],
                    out_hbm.at[c, pl.ds(s * ROWS_PT, ROWS_PT)])

  return _segsum_sc


def kernel(x, cond, edge_index, batch, global_features,
           W1a, b1a, W1b, b1b, W2a, b2a, W2b, b2b):
    edges = edge_index.reshape(2, NCHUNK_TOT, CH)
    batch3 = batch.reshape(NBLK, 1, RB)

    segsum_sc = _make_segsum_sc()
    p1 = _proj_call(x, batch3, cond, global_features, W1a)
    agg1 = segsum_sc(p1, edges)
    p2 = _mid_call(p1, agg1, batch3, cond, global_features, b1a, W1b, b1b, W2a)
    agg2 = segsum_sc(p2, edges)
    return _final_call(p2, agg2, b2a, W2b, b2b)
